# baseline (speedup 1.0000x reference)
import jax
import jax.numpy as jnp
from jax import lax
from jax.experimental import pallas as pl
from jax.experimental.pallas import tpu as pltpu
from jax.experimental.pallas import tpu_sc as plsc

N = 10000
E = 320000
D = 128
H = 32
B = 64

NC = 2
NS = 16
NW = NC * NS
EPW = E // NW
C = 400
NCHUNK = EPW // C
SUB = 50
NSUB = C // SUB
NGRP = C // 16
ZROWS = 640


def _transform_body(x_ref, wl_ref, wr_ref, xl_ref, xr_ref):
    x = x_ref[...]
    xl_ref[...] = jnp.dot(x, wl_ref[...], preferred_element_type=jnp.float32)
    xr_ref[...] = jnp.dot(x, wr_ref[...], preferred_element_type=jnp.float32)


def _edge_body(xl_hbm, xr_hbm, src_hbm, dst_hbm, attb_hbm,
               num_out, den_out,
               src_i, dst_i, xlr, xrr, outr, pbuf, attb, znum, zden,
               sh_num, sh_den, sem):
    c = lax.axis_index("c")
    s = lax.axis_index("s")
    wid = s * NC + c

    pltpu.sync_copy(attb_hbm, attb)

    z16 = jnp.zeros((16,), jnp.float32)

    def zrow(i, carry):
        znum[i, pl.ds(0, 16)] = z16
        znum[i, pl.ds(16, 16)] = z16
        return carry

    lax.fori_loop(0, ZROWS, zrow, 0)

    def zden_row(i, carry):
        zden[pl.ds(i * 16, 16)] = z16
        return carry

    lax.fori_loop(0, N // 16, zden_row, 0)

    @pl.when(s < NS - 1)
    def _():
        pltpu.sync_copy(znum, sh_num.at[pl.ds(s * ZROWS, ZROWS)])

    @pl.when(s == NS - 1)
    def _():
        pltpu.sync_copy(znum.at[pl.ds(0, N - (NS - 1) * ZROWS)],
                        sh_num.at[pl.ds((NS - 1) * ZROWS,
                                        N - (NS - 1) * ZROWS)])

    @pl.when(s == 0)
    def _():
        pltpu.sync_copy(zden, sh_den)

    plsc.subcore_barrier()

    row0 = wid * EPW

    def chunk_body(ci, carry):
        e0 = row0 + ci * C
        pltpu.sync_copy(src_hbm.at[pl.ds(e0, C)], src_i)
        pltpu.sync_copy(dst_hbm.at[pl.ds(e0, C)], dst_i)

        d1 = pltpu.async_copy(xl_hbm.at[src_i], xlr, sem)
        d2 = pltpu.async_copy(xr_hbm.at[dst_i], xrr, sem)
        d1.wait()
        d2.wait()

        slope = jnp.full((16,), 0.2, jnp.float32)
        lanes = lax.iota(jnp.int32, 16)
        hmask = jnp.full((16,), H - 1, jnp.int32)

        @plsc.parallel_loop(0, C, step=16, unroll=2)
        def group_body(e0):
            ev = lanes + jnp.full((16,), e0, jnp.int32)
            acc = jnp.zeros((16,), jnp.float32)
            for k in range(H):
                kv = (lanes + jnp.full((16,), k, jnp.int32)) & hmask
                a = plsc.load_gather(xlr, [ev, kv])
                b = plsc.load_gather(xrr, [ev, kv])
                u = a + b
                lrelu = jnp.maximum(u, slope * u)
                acc = acc + attb[k] * lrelu
            p = jnp.exp(acc)
            pbuf[pl.ds(e0, 16)] = p
            for k in range(H):
                kv = (lanes + jnp.full((16,), k, jnp.int32)) & hmask
                a = plsc.load_gather(xlr, [ev, kv])
                plsc.store_scatter(outr, [ev, kv], p * a)

        pltpu.sync_copy(outr, sh_num.at[dst_i], add=True)
        pltpu.sync_copy(pbuf, sh_den.at[dst_i], add=True)
        return carry

    lax.fori_loop(0, NCHUNK, chunk_body, 0)

    plsc.subcore_barrier()

    @pl.when(s < NS - 1)
    def _():
        pltpu.sync_copy(sh_num.at[pl.ds(s * ZROWS, ZROWS)],
                        num_out.at[c, pl.ds(s * ZROWS, ZROWS)])

    @pl.when(s == NS - 1)
    def _():
        pltpu.sync_copy(
            sh_num.at[pl.ds((NS - 1) * ZROWS, N - (NS - 1) * ZROWS)],
            num_out.at[c, pl.ds((NS - 1) * ZROWS, N - (NS - 1) * ZROWS)])

    @pl.when(s == 0)
    def _():
        pltpu.sync_copy(sh_den, den_out.at[pl.ds(c * N, N)])


def _merge_body(num_ref, den_ref, bias_ref, batch_ref,
                w1_ref, b1_ref, w2_ref, b2_ref, w3_ref, b3_ref, out_ref,
                g_ref):
    num = num_ref[0] + num_ref[1]
    den = den_ref[0] + den_ref[1]
    h = num / (den + 1e-16) + bias_ref[...][None, :]
    batch = batch_ref[...]

    def pool_body(b, carry):
        mask = batch == b
        col = jnp.max(jnp.where(mask, h, -jnp.inf), axis=0)
        g_ref[pl.ds(b, 1), :] = col[None, :]
        return carry

    lax.fori_loop(0, B, pool_body, 0)
    g = g_ref[...]
    g = jnp.where(jnp.isfinite(g), g, 0.0)

    z = jnp.maximum(
        jnp.dot(g, w1_ref[...], preferred_element_type=jnp.float32)
        + b1_ref[...][None, :], 0.0)
    z = jnp.maximum(
        jnp.dot(z, w2_ref[...], preferred_element_type=jnp.float32)
        + b2_ref[...][None, :], 0.0)
    out_ref[...] = (jnp.dot(z, w3_ref[...], preferred_element_type=jnp.float32)
                    + b3_ref[...][None, :])


@jax.jit
def kernel(x, edge_index, batch, Wl, Wr, att, bias, W1, b1, W2, b2, W3, b3):
    xl, xr = pl.pallas_call(
        _transform_body,
        out_shape=[
            jax.ShapeDtypeStruct((N, H), jnp.float32),
            jax.ShapeDtypeStruct((N, H), jnp.float32),
        ],
    )(x, Wl, Wr)

    src = edge_index[0]
    dst = edge_index[1]

    mesh = plsc.VectorSubcoreMesh(core_axis_name="c", subcore_axis_name="s")
    edge_fn = pl.kernel(
        _edge_body,
        out_type=[
            jax.ShapeDtypeStruct((NC, N, H), jnp.float32),
            jax.ShapeDtypeStruct((NC * N,), jnp.float32),
        ],
        mesh=mesh,
        scratch_types=[
            pltpu.VMEM((C,), jnp.int32),
            pltpu.VMEM((C,), jnp.int32),
            pltpu.VMEM((C, H), jnp.float32),
            pltpu.VMEM((C, H), jnp.float32),
            pltpu.VMEM((C, H), jnp.float32),
            pltpu.VMEM((C,), jnp.float32),
            pltpu.VMEM((H, 16), jnp.float32),
            pltpu.VMEM((ZROWS, H), jnp.float32),
            pltpu.VMEM((N,), jnp.float32),
            pltpu.VMEM_SHARED((N, H), jnp.float32),
            pltpu.VMEM_SHARED((N,), jnp.float32),
            pltpu.SemaphoreType.DMA,
        ],
        compiler_params=pltpu.CompilerParams(use_tc_tiling_on_sc=False,
                                             needs_layout_passes=False),
    )
    attb = att[(jnp.arange(H)[:, None] + jnp.arange(16)[None, :]) % H]
    num_part, den_flat = edge_fn(xl, xr, src, dst, attb)
    den_part = den_flat.reshape(NC, N, 1)

    out = pl.pallas_call(
        _merge_body,
        out_shape=jax.ShapeDtypeStruct((B, 4), jnp.float32),
        scratch_shapes=[pltpu.VMEM((B, H), jnp.float32)],
    )(num_part, den_part, bias, batch.reshape(N, 1), W1, b1, W2, b2, W3, b3)
    return out


# speedup vs baseline: 1.0584x; 1.0584x over previous
import jax
import jax.numpy as jnp
from jax import lax
from jax.experimental import pallas as pl
from jax.experimental.pallas import tpu as pltpu
from jax.experimental.pallas import tpu_sc as plsc

N = 10000
E = 320000
D = 128
H = 32
B = 64

NC = 2
NS = 16
NW = NC * NS
EPW = E // NW
C = 400
NCHUNK = EPW // C
SUB = 50
NSUB = C // SUB
NGRP = C // 16
ZROWS = 640


def _transform_body(x_ref, wl_ref, wr_ref, xl_ref, xr_ref):
    x = x_ref[...]
    xl_ref[...] = jnp.dot(x, wl_ref[...], preferred_element_type=jnp.float32)
    xr_ref[...] = jnp.dot(x, wr_ref[...], preferred_element_type=jnp.float32)


def _edge_body(xl_hbm, xr_hbm, src_hbm, dst_hbm, attb_hbm,
               num_out, den_out,
               src_i, dst_i, xlr, xrr, outr, pbuf, attb, znum, zden,
               sh_num, sh_den, sem):
    c = lax.axis_index("c")
    s = lax.axis_index("s")
    wid = s * NC + c

    pltpu.sync_copy(attb_hbm, attb)

    z16 = jnp.zeros((16,), jnp.float32)

    def zrow(i, carry):
        znum[i, pl.ds(0, 16)] = z16
        znum[i, pl.ds(16, 16)] = z16
        return carry

    lax.fori_loop(0, ZROWS, zrow, 0)

    def zden_row(i, carry):
        zden[pl.ds(i * 16, 16)] = z16
        return carry

    lax.fori_loop(0, N // 16, zden_row, 0)

    @pl.when(s < NS - 1)
    def _():
        pltpu.sync_copy(znum, sh_num.at[pl.ds(s * ZROWS, ZROWS)])

    @pl.when(s == NS - 1)
    def _():
        pltpu.sync_copy(znum.at[pl.ds(0, N - (NS - 1) * ZROWS)],
                        sh_num.at[pl.ds((NS - 1) * ZROWS,
                                        N - (NS - 1) * ZROWS)])

    @pl.when(s == 0)
    def _():
        pltpu.sync_copy(zden, sh_den)

    plsc.subcore_barrier()

    pltpu.sync_copy(src_hbm.at[wid], src_i)
    pltpu.sync_copy(dst_hbm.at[wid], dst_i)

    def chunk_body(ci, carry):
        d1 = pltpu.async_copy(xl_hbm.at[src_i.at[ci]], xlr, sem)
        d2 = pltpu.async_copy(xr_hbm.at[dst_i.at[ci]], xrr, sem)
        d1.wait()
        d2.wait()

        slope = jnp.full((16,), 0.2, jnp.float32)
        lanes = lax.iota(jnp.int32, 16)
        hmask = jnp.full((16,), H - 1, jnp.int32)

        @plsc.parallel_loop(0, C, step=16, unroll=2)
        def group_body(e0):
            ev = lanes + jnp.full((16,), e0, jnp.int32)
            acc = jnp.zeros((16,), jnp.float32)
            for k in range(H):
                kv = (lanes + jnp.full((16,), k, jnp.int32)) & hmask
                a = plsc.load_gather(xlr, [ev, kv])
                b = plsc.load_gather(xrr, [ev, kv])
                u = a + b
                lrelu = jnp.maximum(u, slope * u)
                acc = acc + attb[k] * lrelu
            p = jnp.exp(acc)
            pbuf[pl.ds(e0, 16)] = p
            for k in range(H):
                kv = (lanes + jnp.full((16,), k, jnp.int32)) & hmask
                a = plsc.load_gather(xlr, [ev, kv])
                plsc.store_scatter(outr, [ev, kv], p * a)

        pltpu.sync_copy(outr, sh_num.at[dst_i.at[ci]], add=True)
        pltpu.sync_copy(pbuf, sh_den.at[dst_i.at[ci]], add=True)
        return carry

    lax.fori_loop(0, NCHUNK, chunk_body, 0)

    plsc.subcore_barrier()

    @pl.when(s < NS - 1)
    def _():
        pltpu.sync_copy(sh_num.at[pl.ds(s * ZROWS, ZROWS)],
                        num_out.at[c, pl.ds(s * ZROWS, ZROWS)])

    @pl.when(s == NS - 1)
    def _():
        pltpu.sync_copy(
            sh_num.at[pl.ds((NS - 1) * ZROWS, N - (NS - 1) * ZROWS)],
            num_out.at[c, pl.ds((NS - 1) * ZROWS, N - (NS - 1) * ZROWS)])

    @pl.when(s == 0)
    def _():
        pltpu.sync_copy(sh_den, den_out.at[pl.ds(c * N, N)])


def _merge_body(num_ref, den_ref, bias_ref, batch_ref,
                w1_ref, b1_ref, w2_ref, b2_ref, w3_ref, b3_ref, out_ref,
                g_ref):
    num = num_ref[0] + num_ref[1]
    den = den_ref[0] + den_ref[1]
    h = num / (den + 1e-16) + bias_ref[...][None, :]
    batch = batch_ref[...]

    def pool_body(b, carry):
        mask = batch == b
        col = jnp.max(jnp.where(mask, h, -jnp.inf), axis=0)
        g_ref[pl.ds(b, 1), :] = col[None, :]
        return carry

    lax.fori_loop(0, B, pool_body, 0)
    g = g_ref[...]
    g = jnp.where(jnp.isfinite(g), g, 0.0)

    z = jnp.maximum(
        jnp.dot(g, w1_ref[...], preferred_element_type=jnp.float32)
        + b1_ref[...][None, :], 0.0)
    z = jnp.maximum(
        jnp.dot(z, w2_ref[...], preferred_element_type=jnp.float32)
        + b2_ref[...][None, :], 0.0)
    out_ref[...] = (jnp.dot(z, w3_ref[...], preferred_element_type=jnp.float32)
                    + b3_ref[...][None, :])


@jax.jit
def kernel(x, edge_index, batch, Wl, Wr, att, bias, W1, b1, W2, b2, W3, b3):
    xl, xr = pl.pallas_call(
        _transform_body,
        out_shape=[
            jax.ShapeDtypeStruct((N, H), jnp.float32),
            jax.ShapeDtypeStruct((N, H), jnp.float32),
        ],
    )(x, Wl, Wr)

    src = edge_index[0].reshape(NW, NCHUNK, C)
    dst = edge_index[1].reshape(NW, NCHUNK, C)

    mesh = plsc.VectorSubcoreMesh(core_axis_name="c", subcore_axis_name="s")
    edge_fn = pl.kernel(
        _edge_body,
        out_type=[
            jax.ShapeDtypeStruct((NC, N, H), jnp.float32),
            jax.ShapeDtypeStruct((NC * N,), jnp.float32),
        ],
        mesh=mesh,
        scratch_types=[
            pltpu.VMEM((NCHUNK, C), jnp.int32),
            pltpu.VMEM((NCHUNK, C), jnp.int32),
            pltpu.VMEM((C, H), jnp.float32),
            pltpu.VMEM((C, H), jnp.float32),
            pltpu.VMEM((C, H), jnp.float32),
            pltpu.VMEM((C,), jnp.float32),
            pltpu.VMEM((H, 16), jnp.float32),
            pltpu.VMEM((ZROWS, H), jnp.float32),
            pltpu.VMEM((N,), jnp.float32),
            pltpu.VMEM_SHARED((N, H), jnp.float32),
            pltpu.VMEM_SHARED((N,), jnp.float32),
            pltpu.SemaphoreType.DMA,
        ],
        compiler_params=pltpu.CompilerParams(use_tc_tiling_on_sc=False,
                                             needs_layout_passes=False),
    )
    attb = att[(jnp.arange(H)[:, None] + jnp.arange(16)[None, :]) % H]
    num_part, den_flat = edge_fn(xl, xr, src, dst, attb)
    den_part = den_flat.reshape(NC, N, 1)

    out = pl.pallas_call(
        _merge_body,
        out_shape=jax.ShapeDtypeStruct((B, 4), jnp.float32),
        scratch_shapes=[pltpu.VMEM((B, H), jnp.float32)],
    )(num_part, den_part, bias, batch.reshape(N, 1), W1, b1, W2, b2, W3, b3)
    return out


# speedup vs baseline: 1.0996x; 1.0390x over previous
import jax
import jax.numpy as jnp
from jax import lax
from jax.experimental import pallas as pl
from jax.experimental.pallas import tpu as pltpu
from jax.experimental.pallas import tpu_sc as plsc

N = 10000
E = 320000
D = 128
H = 32
B = 64

NC = 2
NS = 16
NW = NC * NS
EPW = E // NW
C = 400
NCHUNK = EPW // C
SUB = 50
NSUB = C // SUB
NGRP = C // 16
ZROWS = 640


def _transform_body(x_ref, wl_ref, wr_ref, xl_ref, xr_ref):
    x = x_ref[...]
    xl_ref[...] = jnp.dot(x, wl_ref[...], preferred_element_type=jnp.float32)
    xr_ref[...] = jnp.dot(x, wr_ref[...], preferred_element_type=jnp.float32)


def _edge_body(xl_hbm, xr_hbm, src_hbm, dst_hbm, attb_hbm,
               num_out, den_out,
               src_i, dst_i, xlr, xrr, outr, pbuf, attb, znum, zden,
               sh_num, sh_den, sem, sem2):
    c = lax.axis_index("c")
    s = lax.axis_index("s")
    wid = s * NC + c

    pltpu.sync_copy(attb_hbm, attb)

    z16 = jnp.zeros((16,), jnp.float32)

    def zrow(i, carry):
        znum[i, pl.ds(0, 16)] = z16
        znum[i, pl.ds(16, 16)] = z16
        return carry

    lax.fori_loop(0, ZROWS, zrow, 0)

    def zden_row(i, carry):
        zden[pl.ds(i * 16, 16)] = z16
        return carry

    lax.fori_loop(0, N // 16, zden_row, 0)

    @pl.when(s < NS - 1)
    def _():
        pltpu.sync_copy(znum, sh_num.at[pl.ds(s * ZROWS, ZROWS)])

    @pl.when(s == NS - 1)
    def _():
        pltpu.sync_copy(znum.at[pl.ds(0, N - (NS - 1) * ZROWS)],
                        sh_num.at[pl.ds((NS - 1) * ZROWS,
                                        N - (NS - 1) * ZROWS)])

    @pl.when(s == 0)
    def _():
        pltpu.sync_copy(zden, sh_den)

    plsc.subcore_barrier()

    pltpu.sync_copy(src_hbm.at[wid], src_i)
    pltpu.sync_copy(dst_hbm.at[wid], dst_i)

    def chunk_body(ci, carry):
        d1 = pltpu.async_copy(xl_hbm.at[src_i.at[ci]], xlr, sem)
        d2 = pltpu.async_copy(xr_hbm.at[dst_i.at[ci]], xrr, sem)

        @pl.when(ci > 0)
        def _():
            pltpu.make_async_copy(outr, sh_num.at[dst_i.at[0]], sem2).wait()
            pltpu.make_async_copy(pbuf, sh_den.at[dst_i.at[0]], sem2).wait()

        d1.wait()
        d2.wait()

        slope = jnp.full((16,), 0.2, jnp.float32)
        lanes = lax.iota(jnp.int32, 16)
        hmask = jnp.full((16,), H - 1, jnp.int32)

        @plsc.parallel_loop(0, C, step=16, unroll=2)
        def group_body(e0):
            ev = lanes + jnp.full((16,), e0, jnp.int32)
            acc = jnp.zeros((16,), jnp.float32)
            for k in range(H):
                kv = (lanes + jnp.full((16,), k, jnp.int32)) & hmask
                a = plsc.load_gather(xlr, [ev, kv])
                b = plsc.load_gather(xrr, [ev, kv])
                u = a + b
                lrelu = jnp.maximum(u, slope * u)
                acc = acc + attb[k] * lrelu
            p = jnp.exp(acc)
            pbuf[pl.ds(e0, 16)] = p
            for k in range(H):
                kv = (lanes + jnp.full((16,), k, jnp.int32)) & hmask
                a = plsc.load_gather(xlr, [ev, kv])
                plsc.store_scatter(outr, [ev, kv], p * a)

        pltpu.async_copy(outr, sh_num.at[dst_i.at[ci]], sem2, add=True)
        pltpu.async_copy(pbuf, sh_den.at[dst_i.at[ci]], sem2, add=True)
        return carry

    lax.fori_loop(0, NCHUNK, chunk_body, 0)

    pltpu.make_async_copy(outr, sh_num.at[dst_i.at[0]], sem2).wait()
    pltpu.make_async_copy(pbuf, sh_den.at[dst_i.at[0]], sem2).wait()

    plsc.subcore_barrier()

    @pl.when(s < NS - 1)
    def _():
        pltpu.sync_copy(sh_num.at[pl.ds(s * ZROWS, ZROWS)],
                        num_out.at[c, pl.ds(s * ZROWS, ZROWS)])

    @pl.when(s == NS - 1)
    def _():
        pltpu.sync_copy(
            sh_num.at[pl.ds((NS - 1) * ZROWS, N - (NS - 1) * ZROWS)],
            num_out.at[c, pl.ds((NS - 1) * ZROWS, N - (NS - 1) * ZROWS)])

    @pl.when(s == 0)
    def _():
        pltpu.sync_copy(sh_den, den_out.at[pl.ds(c * N, N)])


def _merge_body(num_ref, den_ref, bias_ref, batch_ref,
                w1_ref, b1_ref, w2_ref, b2_ref, w3_ref, b3_ref, out_ref,
                g_ref):
    num = num_ref[0] + num_ref[1]
    den = den_ref[0] + den_ref[1]
    h = num / (den + 1e-16) + bias_ref[...][None, :]
    batch = batch_ref[...]

    def pool_body(b, carry):
        mask = batch == b
        col = jnp.max(jnp.where(mask, h, -jnp.inf), axis=0)
        g_ref[pl.ds(b, 1), :] = col[None, :]
        return carry

    lax.fori_loop(0, B, pool_body, 0)
    g = g_ref[...]
    g = jnp.where(jnp.isfinite(g), g, 0.0)

    z = jnp.maximum(
        jnp.dot(g, w1_ref[...], preferred_element_type=jnp.float32)
        + b1_ref[...][None, :], 0.0)
    z = jnp.maximum(
        jnp.dot(z, w2_ref[...], preferred_element_type=jnp.float32)
        + b2_ref[...][None, :], 0.0)
    out_ref[...] = (jnp.dot(z, w3_ref[...], preferred_element_type=jnp.float32)
                    + b3_ref[...][None, :])


@jax.jit
def kernel(x, edge_index, batch, Wl, Wr, att, bias, W1, b1, W2, b2, W3, b3):
    xl, xr = pl.pallas_call(
        _transform_body,
        out_shape=[
            jax.ShapeDtypeStruct((N, H), jnp.float32),
            jax.ShapeDtypeStruct((N, H), jnp.float32),
        ],
    )(x, Wl, Wr)

    src = edge_index[0].reshape(NW, NCHUNK, C)
    dst = edge_index[1].reshape(NW, NCHUNK, C)

    mesh = plsc.VectorSubcoreMesh(core_axis_name="c", subcore_axis_name="s")
    edge_fn = pl.kernel(
        _edge_body,
        out_type=[
            jax.ShapeDtypeStruct((NC, N, H), jnp.float32),
            jax.ShapeDtypeStruct((NC * N,), jnp.float32),
        ],
        mesh=mesh,
        scratch_types=[
            pltpu.VMEM((NCHUNK, C), jnp.int32),
            pltpu.VMEM((NCHUNK, C), jnp.int32),
            pltpu.VMEM((C, H), jnp.float32),
            pltpu.VMEM((C, H), jnp.float32),
            pltpu.VMEM((C, H), jnp.float32),
            pltpu.VMEM((C,), jnp.float32),
            pltpu.VMEM((H, 16), jnp.float32),
            pltpu.VMEM((ZROWS, H), jnp.float32),
            pltpu.VMEM((N,), jnp.float32),
            pltpu.VMEM_SHARED((N, H), jnp.float32),
            pltpu.VMEM_SHARED((N,), jnp.float32),
            pltpu.SemaphoreType.DMA,
            pltpu.SemaphoreType.DMA,
        ],
        compiler_params=pltpu.CompilerParams(use_tc_tiling_on_sc=False,
                                             needs_layout_passes=False),
    )
    attb = att[(jnp.arange(H)[:, None] + jnp.arange(16)[None, :]) % H]
    num_part, den_flat = edge_fn(xl, xr, src, dst, attb)
    den_part = den_flat.reshape(NC, N, 1)

    out = pl.pallas_call(
        _merge_body,
        out_shape=jax.ShapeDtypeStruct((B, 4), jnp.float32),
        scratch_shapes=[pltpu.VMEM((B, H), jnp.float32)],
    )(num_part, den_part, bias, batch.reshape(N, 1), W1, b1, W2, b2, W3, b3)
    return out
